# baseline (speedup 1.0000x reference)
import functools

import jax
import jax.numpy as jnp
from jax import lax
from jax.experimental import pallas as pl
from jax.experimental.pallas import tpu as pltpu
from jax.experimental.pallas import tpu_sc as plsc

_B, _T, _F = 64, 2048, 512
_HF = 256
_ROWS = _B * _T
_NC, _NS = 2, 16
_NW = _NC * _NS
_RPW = _ROWS // _NW
_CHUNK = 32
_NCHUNK = _RPW // _CHUNK

_KS2 = 0x1BD11BDA
_ROT = ((13, 15, 26, 6), (17, 29, 16, 24))


def _u32(v):
    return jnp.uint32(v)


def _rotl(x, r):
    return (x << _u32(r)) | (x >> _u32(32 - r))


def _threefry_bits(lo):
    x0 = jnp.zeros((16,), jnp.uint32)
    x1 = lo
    for i in range(5):
        for r in _ROT[i % 2]:
            x0 = x0 + x1
            x1 = _rotl(x1, r)
            x1 = x1 ^ x0
        ks_a = (0, 0, _KS2)[(i + 1) % 3]
        ks_b = (0, 0, _KS2)[(i + 2) % 3]
        if ks_a:
            x0 = x0 + _u32(ks_a)
        x1 = x1 + _u32(ks_b + i + 1)
    return x0 ^ x1


def _sort_units(un):
    def vs(a, desc):
        sk, _ = plsc.sort_key_val(a, a, descending=desc)
        return sk

    for i in range(16):
        un[i] = vs(un[i], (i & 1) == 1)
    for ku in (2, 4, 8, 16):
        su = ku // 2
        while su >= 1:
            for i in range(16):
                p = i ^ su
                if p > i:
                    mn = jnp.minimum(un[i], un[p])
                    mx = jnp.maximum(un[i], un[p])
                    if (i & ku) == 0:
                        un[i], un[p] = mn, mx
                    else:
                        un[i], un[p] = mx, mn
            su //= 2
        for i in range(16):
            un[i] = vs(un[i], (i & ku) != 0)
    return un


_mesh = plsc.VectorSubcoreMesh(
    core_axis_name="c", subcore_axis_name="s",
    num_cores=_NC, num_subcores=_NS,
)


@functools.partial(
    pl.kernel,
    out_type=jax.ShapeDtypeStruct((_ROWS, _F), jnp.float32),
    mesh=_mesh,
    scratch_types=[
        pltpu.VMEM((_CHUNK, _F), jnp.float32),
        pltpu.VMEM((_CHUNK, _F), jnp.float32),
        pltpu.VMEM((_HF,), jnp.uint32),
    ],
    compiler_params=pltpu.CompilerParams(needs_layout_passes=False),
)
def _sc_permute(x_hbm, out_hbm, xbuf, obuf, kbuf):
    wid = lax.axis_index("s") * _NC + lax.axis_index("c")
    lane = lax.iota(jnp.int32, 16)

    def chunk_body(c, carry):
        row0 = wid * _RPW + c * _CHUNK
        pltpu.sync_copy(x_hbm.at[pl.ds(row0, _CHUNK)], xbuf)

        def row_body(r, rcarry):
            fbase = (row0 + r).astype(jnp.uint32) * _u32(_HF)

            def unit_body(u, ucarry):
                jvec = (u * 16 + lane).astype(jnp.uint32)
                bits = _threefry_bits(fbase + jvec)
                key = ((bits >> _u32(1)) & _u32(0xFFFFFF00)) | jvec
                kbuf[pl.ds(u * 16, 16)] = key
                return ucarry

            lax.fori_loop(0, 16, unit_body, 0)

            un = [kbuf[pl.ds(i * 16, 16)] for i in range(16)]
            un = _sort_units(un)

            rvec = jnp.zeros((16,), jnp.int32) + r
            for i in range(16):
                obuf[r, pl.ds(i * 16, 16)] = xbuf[r, pl.ds(i * 16, 16)]
            for i in range(16):
                idx = (un[i] & _u32(0xFF)).astype(jnp.int32) + _HF
                vals = plsc.load_gather(xbuf, [rvec, idx])
                obuf[r, pl.ds(_HF + i * 16, 16)] = vals
            return rcarry

        lax.fori_loop(0, _CHUNK, row_body, 0)
        pltpu.sync_copy(obuf, out_hbm.at[pl.ds(row0, _CHUNK)])
        return carry

    lax.fori_loop(0, _NCHUNK, chunk_body, 0)


def kernel(x):
    B, T, F = x.shape
    out = _sc_permute(x.reshape(B * T, F))
    return out.reshape(B, T, F)


# speedup vs baseline: 1.0654x; 1.0654x over previous
import functools

import jax
import jax.numpy as jnp
from jax import lax
from jax.experimental import pallas as pl
from jax.experimental.pallas import tpu as pltpu
from jax.experimental.pallas import tpu_sc as plsc

_B, _T, _F = 64, 2048, 512
_HF = 256
_ROWS = _B * _T
_NC, _NS = 2, 16
_NW = _NC * _NS
_RPW = _ROWS // _NW
_CHUNK = 32
_NCHUNK = _RPW // _CHUNK

_KS2 = 0x1BD11BDA
_ROT = ((13, 15, 26, 6), (17, 29, 16, 24))


def _u32(v):
    return jnp.uint32(v)


def _rotl(x, r):
    return (x << _u32(r)) | (x >> _u32(32 - r))


def _threefry_bits(lo):
    x0 = jnp.zeros((16,), jnp.uint32)
    x1 = lo
    for i in range(5):
        for r in _ROT[i % 2]:
            x0 = x0 + x1
            x1 = _rotl(x1, r)
            x1 = x1 ^ x0
        ks_a = (0, 0, _KS2)[(i + 1) % 3]
        ks_b = (0, 0, _KS2)[(i + 2) % 3]
        if ks_a:
            x0 = x0 + _u32(ks_a)
        x1 = x1 + _u32(ks_b + i + 1)
    return x0 ^ x1


def _sort_units(un):
    def vs(a, desc):
        sk, _ = plsc.sort_key_val(a, a, descending=desc)
        return sk

    for i in range(16):
        un[i] = vs(un[i], (i & 1) == 1)
    for ku in (2, 4, 8, 16):
        su = ku // 2
        while su >= 1:
            for i in range(16):
                p = i ^ su
                if p > i:
                    mn = jnp.minimum(un[i], un[p])
                    mx = jnp.maximum(un[i], un[p])
                    if (i & ku) == 0:
                        un[i], un[p] = mn, mx
                    else:
                        un[i], un[p] = mx, mn
            su //= 2
        for i in range(16):
            un[i] = vs(un[i], (i & ku) != 0)
    return un


_mesh = plsc.VectorSubcoreMesh(
    core_axis_name="c", subcore_axis_name="s",
    num_cores=_NC, num_subcores=_NS,
)


@functools.partial(
    pl.kernel,
    out_type=jax.ShapeDtypeStruct((_ROWS, _F), jnp.float32),
    mesh=_mesh,
    scratch_types=[
        pltpu.VMEM((_CHUNK, _F), jnp.float32),
        pltpu.VMEM((_CHUNK, _F), jnp.float32),
    ],
    compiler_params=pltpu.CompilerParams(needs_layout_passes=False),
)
def _sc_permute(x_hbm, out_hbm, xbuf, obuf):
    wid = lax.axis_index("s") * _NC + lax.axis_index("c")
    lane = lax.iota(jnp.int32, 16)

    def chunk_body(c, carry):
        row0 = wid * _RPW + c * _CHUNK
        pltpu.sync_copy(x_hbm.at[pl.ds(row0, _CHUNK)], xbuf)

        def row_body(r, rcarry):
            fbase = (row0 + r).astype(jnp.uint32) * _u32(_HF)

            un = []
            for u in range(16):
                jvec = (u * 16 + lane).astype(jnp.uint32)
                bits = _threefry_bits(fbase + jvec)
                un.append(((bits >> _u32(1)) & _u32(0xFFFFFF00)) | jvec)

            un = _sort_units(un)

            rvec = jnp.zeros((16,), jnp.int32) + r
            for i in range(16):
                obuf[r, pl.ds(i * 16, 16)] = xbuf[r, pl.ds(i * 16, 16)]
            for i in range(16):
                idx = (un[i] & _u32(0xFF)).astype(jnp.int32) + _HF
                vals = plsc.load_gather(xbuf, [rvec, idx])
                obuf[r, pl.ds(_HF + i * 16, 16)] = vals
            return rcarry

        lax.fori_loop(0, _CHUNK, row_body, 0)
        pltpu.sync_copy(obuf, out_hbm.at[pl.ds(row0, _CHUNK)])
        return carry

    lax.fori_loop(0, _NCHUNK, chunk_body, 0)


def kernel(x):
    B, T, F = x.shape
    out = _sc_permute(x.reshape(B * T, F))
    return out.reshape(B, T, F)
